# Initial kernel scaffold; baseline (speedup 1.0000x reference)
import jax
import jax.numpy as jnp
from jax.experimental import pallas as pl


def kernel(x, proto_segments, lt_Wq, lt_bq, lt_Wk, lt_bk, lt_Wv, lt_bv, lt_Wo, lt_bo, st_Wq, st_bq, st_Wk, st_bk, st_Wv, st_bv, st_Wo, st_bo, fusion_W, fusion_b):
    raise NotImplementedError("write your pallas kernel here")


# speedup vs baseline: 1.4521x; 1.4521x over previous
import functools

import jax
import jax.numpy as jnp
from jax.experimental import pallas as pl

S = 2048
D = 768
H = 12
HD = 64
K = 8
SBLK = 256
HPAIR = 2
NEG = -1e9


def _proj_kernel(x_ref, proto_ref,
                 lwq, lbq, lwk, lbk, lwv, lbv,
                 swq, sbq, swk, sbk, swv, sbv,
                 qlt, klt, vlt, qst, kst, vst, o_ref):
    x = x_ref[...]

    def mm(w, b):
        return jax.lax.dot_general(x, w[...], (((1,), (1,)), ((), ())),
                                   preferred_element_type=jnp.float32) + b[...]

    qlt[...] = mm(lwq, lbq)
    klt[...] = mm(lwk, lbk)
    vlt[...] = mm(lwv, lbv)
    qst[...] = mm(swq, sbq)
    kst[...] = mm(swk, sbk)
    vst[...] = mm(swv, sbv)
    sim = jax.lax.dot_general(x, proto_ref[...], (((1,), (1,)), ((), ())),
                              preferred_element_type=jnp.float32)
    seg = jnp.argmax(sim, axis=-1)
    iot = jax.lax.broadcasted_iota(jnp.int32, (SBLK, K), 1)
    o_ref[...] = (iot == seg[:, None].astype(jnp.int32)).astype(jnp.float32)


def _attn_kernel(qlt, klt, vlt, qst, kst, vst, oq_ref, ok_ref,
                 alt_ref, ast_ref):
    oq = oq_ref[...]
    ok = ok_ref[...]
    same = jax.lax.dot_general(oq, ok, (((1,), (1,)), ((), ())),
                               preferred_element_type=jnp.float32)
    scale = 1.0 / (HD ** 0.5)

    for j in range(HPAIR):
        sl = slice(j * HD, (j + 1) * HD)
        q = qlt[:, sl]
        k = klt[:, sl]
        v = vlt[:, sl]
        s = jax.lax.dot_general(q, k, (((1,), (1,)), ((), ())),
                                preferred_element_type=jnp.float32) * scale
        s = s - jnp.max(s, axis=-1, keepdims=True)
        e = jnp.exp(s)
        p = e / jnp.sum(e, axis=-1, keepdims=True)
        alt_ref[:, sl] = jax.lax.dot_general(
            p, v, (((1,), (0,)), ((), ())), preferred_element_type=jnp.float32)
        q = qst[:, sl]
        k = kst[:, sl]
        v = vst[:, sl]
        s = jax.lax.dot_general(q, k, (((1,), (1,)), ((), ())),
                                preferred_element_type=jnp.float32) * scale
        s = jnp.where(same > 0.5, s, NEG)
        s = s - jnp.max(s, axis=-1, keepdims=True)
        e = jnp.exp(s)
        p = e / jnp.sum(e, axis=-1, keepdims=True)
        ast_ref[:, sl] = jax.lax.dot_general(
            p, v, (((1,), (0,)), ((), ())), preferred_element_type=jnp.float32)


def _fuse_kernel(alt_ref, ast_ref, x_ref, oq_ref, ofull_ref,
                 lwo, lbo, swo, sbo, fw, fb, out_ref):
    long_out = jax.lax.dot_general(alt_ref[...], lwo[...],
                                   (((1,), (1,)), ((), ())),
                                   preferred_element_type=jnp.float32) + lbo[...]
    short_out = jax.lax.dot_general(ast_ref[...], swo[...],
                                    (((1,), (1,)), ((), ())),
                                    preferred_element_type=jnp.float32) + sbo[...]
    counts = jnp.sum(ofull_ref[...], axis=0, keepdims=True)
    single = (counts == 1.0).astype(jnp.float32)
    flag = jax.lax.dot_general(oq_ref[...], single,
                               (((1,), (1,)), ((), ())),
                               preferred_element_type=jnp.float32)
    short_out = jnp.where(flag > 0.5, x_ref[...], short_out)
    fwm = fw[...]
    out = jax.lax.dot_general(long_out, fwm[:, :D], (((1,), (1,)), ((), ())),
                              preferred_element_type=jnp.float32)
    out = out + jax.lax.dot_general(short_out, fwm[:, D:],
                                    (((1,), (1,)), ((), ())),
                                    preferred_element_type=jnp.float32)
    out_ref[...] = out + fb[...]


@jax.jit
def _run(x2, proto, lwq, lbq, lwk, lbk, lwv, lbv, lwo, lbo,
         swq, sbq, swk, sbk, swv, sbv, swo, sbo, fw, fb):
    nblk = S // SBLK
    f32 = jnp.float32

    full = lambda shape: pl.BlockSpec(shape, lambda i: (0,) * len(shape))
    sblk = pl.BlockSpec((SBLK, D), lambda i: (i, 0))

    qlt, klt, vlt, qst, kst, vst, onehot = pl.pallas_call(
        _proj_kernel,
        grid=(nblk,),
        in_specs=[sblk, full((K, D)),
                  full((D, D)), full((1, D)), full((D, D)), full((1, D)),
                  full((D, D)), full((1, D)),
                  full((D, D)), full((1, D)), full((D, D)), full((1, D)),
                  full((D, D)), full((1, D))],
        out_specs=[sblk] * 6 + [pl.BlockSpec((SBLK, K), lambda i: (i, 0))],
        out_shape=[jax.ShapeDtypeStruct((S, D), f32)] * 6
        + [jax.ShapeDtypeStruct((S, K), f32)],
    )(x2, proto,
      lwq, lbq.reshape(1, D), lwk, lbk.reshape(1, D), lwv, lbv.reshape(1, D),
      swq, sbq.reshape(1, D), swk, sbk.reshape(1, D), swv, sbv.reshape(1, D))

    nhp = H // HPAIR
    W = HPAIR * HD
    qspec = pl.BlockSpec((SBLK, W), lambda hp, qb: (qb, hp))
    kspec = pl.BlockSpec((S, W), lambda hp, qb: (0, hp))
    oqspec = pl.BlockSpec((SBLK, K), lambda hp, qb: (qb, 0))
    okspec = pl.BlockSpec((S, K), lambda hp, qb: (0, 0))

    alt, ast = pl.pallas_call(
        _attn_kernel,
        grid=(nhp, nblk),
        in_specs=[qspec, kspec, kspec, qspec, kspec, kspec, oqspec, okspec],
        out_specs=[qspec, qspec],
        out_shape=[jax.ShapeDtypeStruct((S, D), f32)] * 2,
    )(qlt, klt, vlt, qst, kst, vst, onehot, onehot)

    out = pl.pallas_call(
        _fuse_kernel,
        grid=(nblk,),
        in_specs=[sblk, sblk, sblk,
                  pl.BlockSpec((SBLK, K), lambda i: (i, 0)), full((S, K)),
                  full((D, D)), full((1, D)), full((D, D)), full((1, D)),
                  full((D, 2 * D)), full((1, D))],
        out_specs=sblk,
        out_shape=jax.ShapeDtypeStruct((S, D), f32),
    )(alt, ast, x2, onehot, onehot,
      lwo, lbo.reshape(1, D), swo, sbo.reshape(1, D), fw, fb.reshape(1, D))
    return out


def kernel(x, proto_segments, lt_Wq, lt_bq, lt_Wk, lt_bk, lt_Wv, lt_bv,
           lt_Wo, lt_bo, st_Wq, st_bq, st_Wk, st_bk, st_Wv, st_bv,
           st_Wo, st_bo, fusion_W, fusion_b):
    x2 = x.reshape(S, D)
    out = _run(x2, proto_segments, lt_Wq, lt_bq, lt_Wk, lt_bk, lt_Wv, lt_bv,
               lt_Wo, lt_bo, st_Wq, st_bq, st_Wk, st_bk, st_Wv, st_bv,
               st_Wo, st_bo, fusion_W, fusion_b)
    return out.reshape(1, S, D)


# speedup vs baseline: 1.4634x; 1.0078x over previous
import functools

import jax
import jax.numpy as jnp
from jax.experimental import pallas as pl

S = 2048
D = 768
H = 12
HD = 64
K = 8
SBLK = 256
HPAIR = 2
NEG = -1e9


def _proj_kernel(x_ref, proto_ref,
                 lwq, lbq, lwk, lbk, lwv, lbv,
                 swq, sbq, swk, sbk, swv, sbv,
                 qlt, klt, vlt, qst, kst, vst, o_ref):
    x = x_ref[...]
    xb = x.astype(jnp.bfloat16)

    def mm(w, b):
        return jax.lax.dot_general(xb, w[...].astype(jnp.bfloat16),
                                   (((1,), (1,)), ((), ())),
                                   preferred_element_type=jnp.float32) + b[...]

    qlt[...] = mm(lwq, lbq)
    klt[...] = mm(lwk, lbk)
    vlt[...] = mm(lwv, lbv)
    qst[...] = mm(swq, sbq)
    kst[...] = mm(swk, sbk)
    vst[...] = mm(swv, sbv)
    sim = jax.lax.dot_general(x, proto_ref[...], (((1,), (1,)), ((), ())),
                              preferred_element_type=jnp.float32)
    seg = jnp.argmax(sim, axis=-1)
    iot = jax.lax.broadcasted_iota(jnp.int32, (SBLK, K), 1)
    o_ref[...] = (iot == seg[:, None].astype(jnp.int32)).astype(jnp.float32)


def _attn_kernel(qlt, klt, vlt, qst, kst, vst, oq_ref, ok_ref,
                 alt_ref, ast_ref):
    oq = oq_ref[...]
    ok = ok_ref[...]
    same = jax.lax.dot_general(oq, ok, (((1,), (1,)), ((), ())),
                               preferred_element_type=jnp.float32)
    scale = 1.0 / (HD ** 0.5)

    bf = jnp.bfloat16
    for j in range(HPAIR):
        sl = slice(j * HD, (j + 1) * HD)
        q = qlt[:, sl].astype(bf)
        k = klt[:, sl].astype(bf)
        v = vlt[:, sl].astype(bf)
        s = jax.lax.dot_general(q, k, (((1,), (1,)), ((), ())),
                                preferred_element_type=jnp.float32) * scale
        s = s - jnp.max(s, axis=-1, keepdims=True)
        e = jnp.exp(s)
        p = (e / jnp.sum(e, axis=-1, keepdims=True)).astype(bf)
        alt_ref[:, sl] = jax.lax.dot_general(
            p, v, (((1,), (0,)), ((), ())), preferred_element_type=jnp.float32)
        q = qst[:, sl].astype(bf)
        k = kst[:, sl].astype(bf)
        v = vst[:, sl].astype(bf)
        s = jax.lax.dot_general(q, k, (((1,), (1,)), ((), ())),
                                preferred_element_type=jnp.float32) * scale
        s = jnp.where(same > 0.5, s, NEG)
        s = s - jnp.max(s, axis=-1, keepdims=True)
        e = jnp.exp(s)
        p = (e / jnp.sum(e, axis=-1, keepdims=True)).astype(bf)
        ast_ref[:, sl] = jax.lax.dot_general(
            p, v, (((1,), (0,)), ((), ())), preferred_element_type=jnp.float32)


def _fuse_kernel(alt_ref, ast_ref, x_ref, oq_ref, ofull_ref,
                 lwo, lbo, swo, sbo, fw, fb, out_ref):
    bf = jnp.bfloat16
    long_out = jax.lax.dot_general(alt_ref[...].astype(bf), lwo[...].astype(bf),
                                   (((1,), (1,)), ((), ())),
                                   preferred_element_type=jnp.float32) + lbo[...]
    short_out = jax.lax.dot_general(ast_ref[...].astype(bf), swo[...].astype(bf),
                                    (((1,), (1,)), ((), ())),
                                    preferred_element_type=jnp.float32) + sbo[...]
    counts = jnp.sum(ofull_ref[...], axis=0, keepdims=True)
    single = (counts == 1.0).astype(jnp.float32)
    flag = jax.lax.dot_general(oq_ref[...], single,
                               (((1,), (1,)), ((), ())),
                               preferred_element_type=jnp.float32)
    short_out = jnp.where(flag > 0.5, x_ref[...], short_out)
    fwm = fw[...].astype(bf)
    out = jax.lax.dot_general(long_out.astype(bf), fwm[:, :D],
                              (((1,), (1,)), ((), ())),
                              preferred_element_type=jnp.float32)
    out = out + jax.lax.dot_general(short_out.astype(bf), fwm[:, D:],
                                    (((1,), (1,)), ((), ())),
                                    preferred_element_type=jnp.float32)
    out_ref[...] = out + fb[...]


@jax.jit
def _run(x2, proto, lwq, lbq, lwk, lbk, lwv, lbv, lwo, lbo,
         swq, sbq, swk, sbk, swv, sbv, swo, sbo, fw, fb):
    nblk = S // SBLK
    f32 = jnp.float32

    full = lambda shape: pl.BlockSpec(shape, lambda i: (0,) * len(shape))
    sblk = pl.BlockSpec((SBLK, D), lambda i: (i, 0))

    qlt, klt, vlt, qst, kst, vst, onehot = pl.pallas_call(
        _proj_kernel,
        grid=(nblk,),
        in_specs=[sblk, full((K, D)),
                  full((D, D)), full((1, D)), full((D, D)), full((1, D)),
                  full((D, D)), full((1, D)),
                  full((D, D)), full((1, D)), full((D, D)), full((1, D)),
                  full((D, D)), full((1, D))],
        out_specs=[sblk] * 6 + [pl.BlockSpec((SBLK, K), lambda i: (i, 0))],
        out_shape=[jax.ShapeDtypeStruct((S, D), f32)] * 6
        + [jax.ShapeDtypeStruct((S, K), f32)],
    )(x2, proto,
      lwq, lbq.reshape(1, D), lwk, lbk.reshape(1, D), lwv, lbv.reshape(1, D),
      swq, sbq.reshape(1, D), swk, sbk.reshape(1, D), swv, sbv.reshape(1, D))

    nhp = H // HPAIR
    W = HPAIR * HD
    qspec = pl.BlockSpec((SBLK, W), lambda hp, qb: (qb, hp))
    kspec = pl.BlockSpec((S, W), lambda hp, qb: (0, hp))
    oqspec = pl.BlockSpec((SBLK, K), lambda hp, qb: (qb, 0))
    okspec = pl.BlockSpec((S, K), lambda hp, qb: (0, 0))

    alt, ast = pl.pallas_call(
        _attn_kernel,
        grid=(nhp, nblk),
        in_specs=[qspec, kspec, kspec, qspec, kspec, kspec, oqspec, okspec],
        out_specs=[qspec, qspec],
        out_shape=[jax.ShapeDtypeStruct((S, D), f32)] * 2,
    )(qlt, klt, vlt, qst, kst, vst, onehot, onehot)

    out = pl.pallas_call(
        _fuse_kernel,
        grid=(nblk,),
        in_specs=[sblk, sblk, sblk,
                  pl.BlockSpec((SBLK, K), lambda i: (i, 0)), full((S, K)),
                  full((D, D)), full((1, D)), full((D, D)), full((1, D)),
                  full((D, 2 * D)), full((1, D))],
        out_specs=sblk,
        out_shape=jax.ShapeDtypeStruct((S, D), f32),
    )(alt, ast, x2, onehot, onehot,
      lwo, lbo.reshape(1, D), swo, sbo.reshape(1, D), fw, fb.reshape(1, D))
    return out


def kernel(x, proto_segments, lt_Wq, lt_bq, lt_Wk, lt_bk, lt_Wv, lt_bv,
           lt_Wo, lt_bo, st_Wq, st_bq, st_Wk, st_bk, st_Wv, st_bv,
           st_Wo, st_bo, fusion_W, fusion_b):
    x2 = x.reshape(S, D)
    out = _run(x2, proto_segments, lt_Wq, lt_bq, lt_Wk, lt_bk, lt_Wv, lt_bv,
               lt_Wo, lt_bo, st_Wq, st_bq, st_Wk, st_bk, st_Wv, st_bv,
               st_Wo, st_bo, fusion_W, fusion_b)
    return out.reshape(1, S, D)


# speedup vs baseline: 2.2152x; 1.5138x over previous
import jax
import jax.numpy as jnp
from jax.experimental import pallas as pl

S = 2048
D = 768
H = 12
HD = 64
K = 8
SBLK = 256
HPAIR = 2
SCALE = 0.125


def _proj_kernel(x_ref, proto_ref,
                 lwq, lbq, lwk, lbk, lwv, lbv,
                 swq, sbq, swk, sbk, swv, sbv,
                 qlt, klt, vlt, qst, kst, vst, o_ref):
    x = x_ref[...]
    xb = x.astype(jnp.bfloat16)

    def mm(w, b, scl):
        r = jax.lax.dot_general(xb, w[...].astype(jnp.bfloat16),
                                (((1,), (1,)), ((), ())),
                                preferred_element_type=jnp.float32) + b[...]
        return (r * scl).astype(jnp.bfloat16)

    qlt[...] = mm(lwq, lbq, SCALE)
    klt[...] = mm(lwk, lbk, 1.0)
    vlt[...] = mm(lwv, lbv, 1.0)
    qst[...] = mm(swq, sbq, SCALE)
    kst[...] = mm(swk, sbk, 1.0)
    vst[...] = mm(swv, sbv, 1.0)
    sim = jax.lax.dot_general(x, proto_ref[...], (((1,), (1,)), ((), ())),
                              preferred_element_type=jnp.float32)
    seg = jnp.argmax(sim, axis=-1)
    iot = jax.lax.broadcasted_iota(jnp.int32, (SBLK, K), 1)
    o_ref[...] = (iot == seg[:, None].astype(jnp.int32)).astype(jnp.float32)


def _attn_kernel(qlt, klt, vlt, qst, kst, vst, oq_ref, ok_ref,
                 alt_ref, ast_ref):
    oq = oq_ref[...]
    ok = ok_ref[...]
    same = jax.lax.dot_general(oq, ok, (((1,), (1,)), ((), ())),
                               preferred_element_type=jnp.float32)

    def attend(q_full, k_full, v_full, sl, mask):
        q = q_full[:, sl]
        k = k_full[:, sl]
        v = v_full[:, sl]
        s = jax.lax.dot_general(q, k, (((1,), (1,)), ((), ())),
                                preferred_element_type=jnp.float32)
        e = jnp.exp(s)
        if mask is not None:
            e = e * mask
        z = jnp.sum(e, axis=-1, keepdims=True)
        eb = e.astype(jnp.bfloat16)
        av = jax.lax.dot_general(eb, v, (((1,), (0,)), ((), ())),
                                 preferred_element_type=jnp.float32)
        return (av * (1.0 / z)).astype(jnp.bfloat16)

    for j in range(HPAIR):
        sl = slice(j * HD, (j + 1) * HD)
        alt_ref[:, sl] = attend(qlt, klt, vlt, sl, None)
        ast_ref[:, sl] = attend(qst, kst, vst, sl, same)


def _fuse_kernel(alt_ref, ast_ref, x_ref, oq_ref, ofull_ref,
                 lwo, lbo, swo, sbo, fw, fb, out_ref):
    bf = jnp.bfloat16
    long_out = jax.lax.dot_general(alt_ref[...], lwo[...].astype(bf),
                                   (((1,), (1,)), ((), ())),
                                   preferred_element_type=jnp.float32) + lbo[...]
    short_out = jax.lax.dot_general(ast_ref[...], swo[...].astype(bf),
                                    (((1,), (1,)), ((), ())),
                                    preferred_element_type=jnp.float32) + sbo[...]
    counts = jnp.sum(ofull_ref[...], axis=0, keepdims=True)
    single = (counts == 1.0).astype(jnp.float32)
    flag = jax.lax.dot_general(oq_ref[...], single,
                               (((1,), (1,)), ((), ())),
                               preferred_element_type=jnp.float32)
    short_out = jnp.where(flag > 0.5, x_ref[...], short_out)
    fwm = fw[...].astype(bf)
    out = jax.lax.dot_general(long_out.astype(bf), fwm[:, :D],
                              (((1,), (1,)), ((), ())),
                              preferred_element_type=jnp.float32)
    out = out + jax.lax.dot_general(short_out.astype(bf), fwm[:, D:],
                                    (((1,), (1,)), ((), ())),
                                    preferred_element_type=jnp.float32)
    out_ref[...] = out + fb[...]


@jax.jit
def _run(x2, proto, lwq, lbq, lwk, lbk, lwv, lbv, lwo, lbo,
         swq, sbq, swk, sbk, swv, sbv, swo, sbo, fw, fb):
    nblk = S // SBLK
    f32 = jnp.float32
    bf16 = jnp.bfloat16

    full = lambda shape: pl.BlockSpec(shape, lambda i: (0,) * len(shape))
    sblk = pl.BlockSpec((SBLK, D), lambda i: (i, 0))

    qlt, klt, vlt, qst, kst, vst, onehot = pl.pallas_call(
        _proj_kernel,
        grid=(nblk,),
        in_specs=[sblk, full((K, D)),
                  full((D, D)), full((1, D)), full((D, D)), full((1, D)),
                  full((D, D)), full((1, D)),
                  full((D, D)), full((1, D)), full((D, D)), full((1, D)),
                  full((D, D)), full((1, D))],
        out_specs=[sblk] * 6 + [pl.BlockSpec((SBLK, K), lambda i: (i, 0))],
        out_shape=[jax.ShapeDtypeStruct((S, D), bf16)] * 6
        + [jax.ShapeDtypeStruct((S, K), f32)],
    )(x2, proto,
      lwq, lbq.reshape(1, D), lwk, lbk.reshape(1, D), lwv, lbv.reshape(1, D),
      swq, sbq.reshape(1, D), swk, sbk.reshape(1, D), swv, sbv.reshape(1, D))

    nhp = H // HPAIR
    W = HPAIR * HD
    qspec = pl.BlockSpec((SBLK, W), lambda hp, qb: (qb, hp))
    kspec = pl.BlockSpec((S, W), lambda hp, qb: (0, hp))
    oqspec = pl.BlockSpec((SBLK, K), lambda hp, qb: (qb, 0))
    okspec = pl.BlockSpec((S, K), lambda hp, qb: (0, 0))

    alt, ast = pl.pallas_call(
        _attn_kernel,
        grid=(nhp, nblk),
        in_specs=[qspec, kspec, kspec, qspec, kspec, kspec, oqspec, okspec],
        out_specs=[qspec, qspec],
        out_shape=[jax.ShapeDtypeStruct((S, D), bf16)] * 2,
    )(qlt, klt, vlt, qst, kst, vst, onehot, onehot)

    out = pl.pallas_call(
        _fuse_kernel,
        grid=(nblk,),
        in_specs=[sblk, sblk, sblk,
                  pl.BlockSpec((SBLK, K), lambda i: (i, 0)), full((S, K)),
                  full((D, D)), full((1, D)), full((D, D)), full((1, D)),
                  full((D, 2 * D)), full((1, D))],
        out_specs=sblk,
        out_shape=jax.ShapeDtypeStruct((S, D), f32),
    )(alt, ast, x2, onehot, onehot,
      lwo, lbo.reshape(1, D), swo, sbo.reshape(1, D), fw, fb.reshape(1, D))
    return out


def kernel(x, proto_segments, lt_Wq, lt_bq, lt_Wk, lt_bk, lt_Wv, lt_bv,
           lt_Wo, lt_bo, st_Wq, st_bq, st_Wk, st_bk, st_Wv, st_bv,
           st_Wo, st_bo, fusion_W, fusion_b):
    x2 = x.reshape(S, D)
    out = _run(x2, proto_segments, lt_Wq, lt_bq, lt_Wk, lt_bk, lt_Wv, lt_bv,
               lt_Wo, lt_bo, st_Wq, st_bq, st_Wk, st_bk, st_Wv, st_bv,
               st_Wo, st_bo, fusion_W, fusion_b)
    return out.reshape(1, S, D)


# speedup vs baseline: 2.4887x; 1.1235x over previous
import jax
import jax.numpy as jnp
from jax.experimental import pallas as pl

S = 2048
D = 768
H = 12
HD = 64
K = 8
SBLK = 256
ABLK = 512
HPAIR = 4
SCALE = 0.125


def _proj_kernel(x_ref, proto_ref,
                 lwq, lbq, lwk, lbk, lwv, lbv,
                 swq, sbq, swk, sbk, swv, sbv,
                 qlt, klt, vlt, qst, kst, vst, o_ref):
    x = x_ref[...]
    xb = x.astype(jnp.bfloat16)

    def mm(w, b, scl):
        r = jax.lax.dot_general(xb, w[...].astype(jnp.bfloat16),
                                (((1,), (1,)), ((), ())),
                                preferred_element_type=jnp.float32) + b[...]
        return (r * scl).astype(jnp.bfloat16)

    qlt[...] = mm(lwq, lbq, SCALE)
    klt[...] = mm(lwk, lbk, 1.0)
    vlt[...] = mm(lwv, lbv, 1.0)
    qst[...] = mm(swq, sbq, SCALE)
    kst[...] = mm(swk, sbk, 1.0)
    vst[...] = mm(swv, sbv, 1.0)
    sim = jax.lax.dot_general(x, proto_ref[...], (((1,), (1,)), ((), ())),
                              preferred_element_type=jnp.float32)
    seg = jnp.argmax(sim, axis=-1)
    iot = jax.lax.broadcasted_iota(jnp.int32, (SBLK, K), 1)
    o_ref[...] = (iot == seg[:, None].astype(jnp.int32)).astype(jnp.float32)


def _attn_kernel(qlt, klt, vlt, qst, kst, vst, oq_ref, ok_ref,
                 alt_ref, ast_ref):
    oq = oq_ref[...]
    ok = ok_ref[...]
    same = jax.lax.dot_general(oq, ok, (((1,), (1,)), ((), ())),
                               preferred_element_type=jnp.float32)

    def attend(q_full, k_full, v_full, sl, mask):
        q = q_full[:, sl]
        k = k_full[:, sl]
        v = v_full[:, sl]
        s = jax.lax.dot_general(q, k, (((1,), (1,)), ((), ())),
                                preferred_element_type=jnp.float32)
        e = jnp.exp(s)
        if mask is not None:
            e = e * mask
        z = jnp.sum(e, axis=-1, keepdims=True)
        eb = e.astype(jnp.bfloat16)
        av = jax.lax.dot_general(eb, v, (((1,), (0,)), ((), ())),
                                 preferred_element_type=jnp.float32)
        return (av * (1.0 / z)).astype(jnp.bfloat16)

    for j in range(HPAIR):
        sl = slice(j * HD, (j + 1) * HD)
        alt_ref[:, sl] = attend(qlt, klt, vlt, sl, None)
        ast_ref[:, sl] = attend(qst, kst, vst, sl, same)


def _fuse_kernel(alt_ref, ast_ref, x_ref, oq_ref, ofull_ref,
                 lwo, lbo, swo, sbo, fw, fb, out_ref):
    bf = jnp.bfloat16
    long_out = jax.lax.dot_general(alt_ref[...], lwo[...].astype(bf),
                                   (((1,), (1,)), ((), ())),
                                   preferred_element_type=jnp.float32) + lbo[...]
    short_out = jax.lax.dot_general(ast_ref[...], swo[...].astype(bf),
                                    (((1,), (1,)), ((), ())),
                                    preferred_element_type=jnp.float32) + sbo[...]
    counts = jnp.sum(ofull_ref[...], axis=0, keepdims=True)
    single = (counts == 1.0).astype(jnp.float32)
    flag = jax.lax.dot_general(oq_ref[...], single,
                               (((1,), (1,)), ((), ())),
                               preferred_element_type=jnp.float32)
    short_out = jnp.where(flag > 0.5, x_ref[...], short_out)
    fwm = fw[...].astype(bf)
    out = jax.lax.dot_general(long_out.astype(bf), fwm[:, :D],
                              (((1,), (1,)), ((), ())),
                              preferred_element_type=jnp.float32)
    out = out + jax.lax.dot_general(short_out.astype(bf), fwm[:, D:],
                                    (((1,), (1,)), ((), ())),
                                    preferred_element_type=jnp.float32)
    out_ref[...] = out + fb[...]


@jax.jit
def _run(x2, proto, lwq, lbq, lwk, lbk, lwv, lbv, lwo, lbo,
         swq, sbq, swk, sbk, swv, sbv, swo, sbo, fw, fb):
    nblk = S // SBLK
    f32 = jnp.float32
    bf16 = jnp.bfloat16

    full = lambda shape: pl.BlockSpec(shape, lambda i: (0,) * len(shape))
    sblk = pl.BlockSpec((SBLK, D), lambda i: (i, 0))

    qlt, klt, vlt, qst, kst, vst, onehot = pl.pallas_call(
        _proj_kernel,
        grid=(nblk,),
        in_specs=[sblk, full((K, D)),
                  full((D, D)), full((1, D)), full((D, D)), full((1, D)),
                  full((D, D)), full((1, D)),
                  full((D, D)), full((1, D)), full((D, D)), full((1, D)),
                  full((D, D)), full((1, D))],
        out_specs=[sblk] * 6 + [pl.BlockSpec((SBLK, K), lambda i: (i, 0))],
        out_shape=[jax.ShapeDtypeStruct((S, D), bf16)] * 6
        + [jax.ShapeDtypeStruct((S, K), f32)],
    )(x2, proto,
      lwq, lbq.reshape(1, D), lwk, lbk.reshape(1, D), lwv, lbv.reshape(1, D),
      swq, sbq.reshape(1, D), swk, sbk.reshape(1, D), swv, sbv.reshape(1, D))

    nhp = H // HPAIR
    W = HPAIR * HD
    qspec = pl.BlockSpec((ABLK, W), lambda hp, qb: (qb, hp))
    kspec = pl.BlockSpec((S, W), lambda hp, qb: (0, hp))
    oqspec = pl.BlockSpec((ABLK, K), lambda hp, qb: (qb, 0))
    okspec = pl.BlockSpec((S, K), lambda hp, qb: (0, 0))

    alt, ast = pl.pallas_call(
        _attn_kernel,
        grid=(nhp, S // ABLK),
        in_specs=[qspec, kspec, kspec, qspec, kspec, kspec, oqspec, okspec],
        out_specs=[qspec, qspec],
        out_shape=[jax.ShapeDtypeStruct((S, D), bf16)] * 2,
    )(qlt, klt, vlt, qst, kst, vst, onehot, onehot)

    out = pl.pallas_call(
        _fuse_kernel,
        grid=(nblk,),
        in_specs=[sblk, sblk, sblk,
                  pl.BlockSpec((SBLK, K), lambda i: (i, 0)), full((S, K)),
                  full((D, D)), full((1, D)), full((D, D)), full((1, D)),
                  full((D, 2 * D)), full((1, D))],
        out_specs=sblk,
        out_shape=jax.ShapeDtypeStruct((S, D), f32),
    )(alt, ast, x2, onehot, onehot,
      lwo, lbo.reshape(1, D), swo, sbo.reshape(1, D), fw, fb.reshape(1, D))
    return out


def kernel(x, proto_segments, lt_Wq, lt_bq, lt_Wk, lt_bk, lt_Wv, lt_bv,
           lt_Wo, lt_bo, st_Wq, st_bq, st_Wk, st_bk, st_Wv, st_bv,
           st_Wo, st_bo, fusion_W, fusion_b):
    x2 = x.reshape(S, D)
    out = _run(x2, proto_segments, lt_Wq, lt_bq, lt_Wk, lt_bk, lt_Wv, lt_bv,
               lt_Wo, lt_bo, st_Wq, st_bq, st_Wk, st_bk, st_Wv, st_bv,
               st_Wo, st_bo, fusion_W, fusion_b)
    return out.reshape(1, S, D)
